# baseline (speedup 1.0000x reference)
import jax
import jax.numpy as jnp
from jax import lax
from jax.experimental import pallas as pl
from jax.experimental.pallas import tpu as pltpu
from jax.experimental.pallas import tpu_sc as plsc

B = 16
CLS = 100
POOL = 100
CTX_LEN = 12
D = 512
SEQ = 77
SUF = SEQ - 1 - CTX_LEN * 2
NC_SUF = SEQ - 1 - CTX_LEN
NC_SEQ = 1 + 2 * CTX_LEN + NC_SUF

CB = 50
NCB = CLS // CB


def _prompts_body(ig_ref, ia_ref, g0, g1, a0, a1, pre, suf, out):
    s = pl.program_id(0)
    b = s // NCB
    cb = s % NCB
    use_g = b < 8
    row0 = jnp.where(use_g, g0[...], a0[...])
    row1 = jnp.where(use_g, g1[...], a1[...])
    out[:, 0:1, :] = pre[...]
    out[:, 1:1 + CTX_LEN, :] = jnp.broadcast_to(row0, (CB, CTX_LEN, D))
    out[:, 1 + CTX_LEN:1 + 2 * CTX_LEN, :] = jnp.broadcast_to(
        row1, (CB, CTX_LEN, D))
    out[:, 1 + 2 * CTX_LEN:SEQ, :] = suf[pl.ds(cb * CB, CB)]


def _build_prompts_call():
    def g0_map(s, ig, ia):
        b = s // NCB
        return jnp.where(b < 8, ig[2 * b], 0), 0, 0

    def g1_map(s, ig, ia):
        b = s // NCB
        return jnp.where(b < 8, ig[2 * b + 1], 0), 0, 0

    def a0_map(s, ig, ia):
        b = s // NCB
        return jnp.where(b >= 8, ia[(2 * b - 16) % B], 0), 0, 0

    def a1_map(s, ig, ia):
        b = s // NCB
        return jnp.where(b >= 8, ia[(2 * b - 15) % B], 0), 0, 0

    grid_spec = pltpu.PrefetchScalarGridSpec(
        num_scalar_prefetch=2,
        grid=(B * NCB,),
        in_specs=[
            pl.BlockSpec((1, CTX_LEN, D), g0_map),
            pl.BlockSpec((1, CTX_LEN, D), g1_map),
            pl.BlockSpec((1, CTX_LEN, D), a0_map),
            pl.BlockSpec((1, CTX_LEN, D), a1_map),
            pl.BlockSpec((CB, 1, D), lambda s, ig, ia: (s % NCB, 0, 0)),
            pl.BlockSpec((CLS, SUF, D), lambda s, ig, ia: (0, 0, 0)),
        ],
        out_specs=pl.BlockSpec((CB, SEQ, D),
                               lambda s, ig, ia: (s, 0, 0)),
    )
    return pl.pallas_call(
        _prompts_body,
        grid_spec=grid_spec,
        out_shape=jax.ShapeDtypeStruct((B * CLS, SEQ, D), jnp.float32),
    )


def _row_copies(row, ncp_v, ncs_v, gp, ap, nc_out, sem):
    return [
        pltpu.async_copy(ncp_v, nc_out.at[row, pl.ds(0, 1), :], sem),
        pltpu.async_copy(gp.at[row], nc_out.at[row, pl.ds(1, CTX_LEN), :],
                         sem),
        pltpu.async_copy(ap.at[row],
                         nc_out.at[row, pl.ds(1 + CTX_LEN, CTX_LEN), :], sem),
        pltpu.async_copy(ncs_v,
                         nc_out.at[row, pl.ds(1 + 2 * CTX_LEN, NC_SUF), :],
                         sem),
    ]


def _sc_nc_body(ncp, ncs, gp, ap, nctok, tokp,
                nc_out, nc_tok_out, tok_out,
                ncp_v, ncs_v, nctok_v, tokp_v, sem):
    c = lax.axis_index("c")
    sid = lax.axis_index("s")
    wid = sid * 2 + c

    @pl.when(wid < 12)
    def _():
        pltpu.sync_copy(ncp, ncp_v)
        pltpu.sync_copy(ncs, ncs_v)
        cps = []
        for j in range(8):
            cps += _row_copies(wid * 8 + j, ncp_v, ncs_v, gp, ap, nc_out, sem)
        for cp in cps:
            cp.wait()

    @pl.when(wid == 12)
    def _():
        pltpu.sync_copy(ncp, ncp_v)
        pltpu.sync_copy(ncs, ncs_v)
        cps = []
        for j in range(4):
            cps += _row_copies(96 + j, ncp_v, ncs_v, gp, ap, nc_out, sem)
        for cp in cps:
            cp.wait()

    @pl.when(wid == 13)
    def _():
        pltpu.sync_copy(tokp, tokp_v)
        cps = [pltpu.async_copy(tokp_v, tok_out.at[pl.ds(b * CLS, CLS)], sem)
               for b in range(B)]
        for cp in cps:
            cp.wait()

    @pl.when(wid == 14)
    def _():
        pltpu.sync_copy(nctok, nctok_v)
        cps = [pltpu.async_copy(nctok_v, nc_tok_out.at[pl.ds(r, 1)], sem)
               for r in range(POOL)]
        for cp in cps:
            cp.wait()


def _sc_nc(ncp, ncs, gp, ap, nctok, tokp):
    return pl.kernel(
        _sc_nc_body,
        out_type=(
            jax.ShapeDtypeStruct((POOL, NC_SEQ, D), jnp.float32),
            jax.ShapeDtypeStruct((POOL, SEQ), jnp.int32),
            jax.ShapeDtypeStruct((B * CLS, SEQ), jnp.int32),
        ),
        mesh=plsc.VectorSubcoreMesh(core_axis_name="c", subcore_axis_name="s"),
        compiler_params=pltpu.CompilerParams(use_tc_tiling_on_sc=False),
        scratch_types=[
            pltpu.VMEM((1, D), jnp.float32),
            pltpu.VMEM((NC_SUF, D), jnp.float32),
            pltpu.VMEM((1, SEQ), jnp.int32),
            pltpu.VMEM((CLS, SEQ), jnp.int32),
            pltpu.SemaphoreType.DMA,
        ],
    )(ncp, ncs, gp, ap, nctok, tokp)


def kernel(indices_g, indices_a, global_prompt, attribute_prompt,
           token_prefix, token_suffix, nc_token_prefix, nc_token_suffix,
           tokenized_prompts, nc_tokenized_prompts):
    ig = indices_g.astype(jnp.int32)
    ia = indices_a.astype(jnp.int32)
    tokp = tokenized_prompts.astype(jnp.int32)
    nctok = nc_tokenized_prompts.astype(jnp.int32)

    prompts = _build_prompts_call()(
        ig, ia, global_prompt, global_prompt, attribute_prompt,
        attribute_prompt, token_prefix, token_suffix)

    nc_prompts, nc_tok, tok = _sc_nc(
        nc_token_prefix.reshape(1, D), nc_token_suffix.reshape(NC_SUF, D),
        global_prompt, attribute_prompt, nctok, tokp)

    return (prompts, tok, nc_prompts, nc_tok)


# speedup vs baseline: 1.4105x; 1.4105x over previous
import jax
import jax.numpy as jnp
from jax import lax
from jax.experimental import pallas as pl
from jax.experimental.pallas import tpu as pltpu
from jax.experimental.pallas import tpu_sc as plsc

B = 16
CLS = 100
POOL = 100
CTX_LEN = 12
D = 512
SEQ = 77
SUF = SEQ - 1 - CTX_LEN * 2
NC_SUF = SEQ - 1 - CTX_LEN
NC_SEQ = 1 + 2 * CTX_LEN + NC_SUF

CB = 50
NCB = CLS // CB


def _sc_gather_body(ig, ia, gp, ap, out, idx_v, rows_v, sem):
    c = lax.axis_index("c")
    s = lax.axis_index("s")
    wid = s * 2 + c

    @pl.when(wid == 0)
    def _():
        pltpu.sync_copy(ig, idx_v)
        pltpu.async_copy(gp.at[idx_v], rows_v, sem).wait()
        cps = [pltpu.async_copy(rows_v.at[i], out.at[i // 2, i % 2], sem)
               for i in range(B)]
        for cp in cps:
            cp.wait()

    @pl.when(wid == 1)
    def _():
        pltpu.sync_copy(ia, idx_v)
        pltpu.async_copy(ap.at[idx_v], rows_v, sem).wait()
        cps = [pltpu.async_copy(rows_v.at[i], out.at[8 + i // 2, i % 2], sem)
               for i in range(B)]
        for cp in cps:
            cp.wait()


def _sc_gather(ig, ia, gp, ap):
    return pl.kernel(
        _sc_gather_body,
        out_type=jax.ShapeDtypeStruct((B, 2, CTX_LEN, D), jnp.float32),
        mesh=plsc.VectorSubcoreMesh(core_axis_name="c", subcore_axis_name="s"),
        compiler_params=pltpu.CompilerParams(use_tc_tiling_on_sc=False),
        scratch_types=[
            pltpu.VMEM((B,), jnp.int32),
            pltpu.VMEM((B, CTX_LEN, D), jnp.float32),
            pltpu.SemaphoreType.DMA,
        ],
    )(ig, ia, gp, ap)


NBUF = 3
NSTEP = B * NCB


def _assemble(buf, ctx, pre, suf, cb):
    buf[:, 0:1, :] = pre[...]
    buf[:, 1:1 + CTX_LEN, :] = jnp.broadcast_to(ctx[0, 0], (CB, CTX_LEN, D))
    buf[:, 1 + CTX_LEN:1 + 2 * CTX_LEN, :] = jnp.broadcast_to(
        ctx[0, 1], (CB, CTX_LEN, D))
    buf[:, 1 + 2 * CTX_LEN:SEQ, :] = suf[pl.ds(cb * CB, CB)]


def _prompts_body(ctx, pre, suf, out, bufs, sems):
    s = pl.program_id(0)
    cb = s % NCB
    i = s % NBUF

    @pl.when(s >= NBUF)
    def _():
        pltpu.make_async_copy(
            bufs.at[i], out.at[pl.ds((s - NBUF) * CB, CB)], sems.at[i]
        ).wait()

    _assemble(bufs.at[i], ctx, pre, suf, cb)
    pltpu.make_async_copy(
        bufs.at[i], out.at[pl.ds(s * CB, CB)], sems.at[i]).start()

    @pl.when(s == NSTEP - 1)
    def _():
        for k in range(NBUF):
            t = NSTEP - NBUF + k
            pltpu.make_async_copy(
                bufs.at[t % NBUF], out.at[pl.ds(t * CB, CB)],
                sems.at[t % NBUF]).wait()


def _build_prompts_call():
    return pl.pallas_call(
        _prompts_body,
        grid=(NSTEP,),
        in_specs=[
            pl.BlockSpec((1, 2, CTX_LEN, D), lambda s: (s // NCB, 0, 0, 0)),
            pl.BlockSpec((CB, 1, D), lambda s: (s % NCB, 0, 0)),
            pl.BlockSpec((CLS, SUF, D), lambda s: (0, 0, 0)),
        ],
        out_specs=pl.BlockSpec(memory_space=pltpu.MemorySpace.HBM),
        out_shape=jax.ShapeDtypeStruct((B * CLS, SEQ, D), jnp.float32),
        scratch_shapes=[
            pltpu.VMEM((NBUF, CB, SEQ, D), jnp.float32),
            pltpu.SemaphoreType.DMA((NBUF,)),
        ],
    )


def _nc_body(ncp, gp, ap, ncs, nctok, tokp, out, nc_tok_out, tok_out):
    out[:, 0:1, :] = jnp.broadcast_to(ncp[...], (CB, 1, D))
    out[:, 1:1 + CTX_LEN, :] = gp[...]
    out[:, 1 + CTX_LEN:1 + 2 * CTX_LEN, :] = ap[...]
    out[:, 1 + 2 * CTX_LEN:NC_SEQ, :] = jnp.broadcast_to(
        ncs[...], (CB, NC_SUF, D))
    nc_tok_out[...] = jnp.broadcast_to(nctok[...], (POOL, SEQ))
    t = tokp[...]
    for b in range(B):
        tok_out[pl.ds(b * CLS, CLS), :] = t


def _build_nc_call():
    return pl.pallas_call(
        _nc_body,
        grid=(NCB,),
        in_specs=[
            pl.BlockSpec((1, 1, D), lambda i: (0, 0, 0)),
            pl.BlockSpec((CB, CTX_LEN, D), lambda i: (i, 0, 0)),
            pl.BlockSpec((CB, CTX_LEN, D), lambda i: (i, 0, 0)),
            pl.BlockSpec((1, NC_SUF, D), lambda i: (0, 0, 0)),
            pl.BlockSpec((1, SEQ), lambda i: (0, 0)),
            pl.BlockSpec((CLS, SEQ), lambda i: (0, 0)),
        ],
        out_specs=[
            pl.BlockSpec((CB, NC_SEQ, D), lambda i: (i, 0, 0)),
            pl.BlockSpec((POOL, SEQ), lambda i: (0, 0)),
            pl.BlockSpec((B * CLS, SEQ), lambda i: (0, 0)),
        ],
        out_shape=[
            jax.ShapeDtypeStruct((POOL, NC_SEQ, D), jnp.float32),
            jax.ShapeDtypeStruct((POOL, SEQ), jnp.int32),
            jax.ShapeDtypeStruct((B * CLS, SEQ), jnp.int32),
        ],
    )


def kernel(indices_g, indices_a, global_prompt, attribute_prompt,
           token_prefix, token_suffix, nc_token_prefix, nc_token_suffix,
           tokenized_prompts, nc_tokenized_prompts):
    ig = indices_g.astype(jnp.int32)
    ia = indices_a.astype(jnp.int32)
    tokp = tokenized_prompts.astype(jnp.int32)
    nctok = nc_tokenized_prompts.astype(jnp.int32)

    ctx = _sc_gather(ig, ia, global_prompt, attribute_prompt)

    prompts = _build_prompts_call()(ctx, token_prefix, token_suffix)

    nc_prompts, nc_tok, tok = _build_nc_call()(
        nc_token_prefix, global_prompt, attribute_prompt,
        nc_token_suffix, nctok, tokp)

    return (prompts, tok, nc_prompts, nc_tok)
